# baseline (speedup 1.0000x reference)
import functools

import jax
import jax.numpy as jnp
from jax import lax
from jax.experimental import pallas as pl
from jax.experimental.pallas import tpu as pltpu
from jax.experimental.pallas import tpu_sc as plsc

B = 16384
H = 64
NT = 4
CH = 128
BM = 512

_HI = jax.lax.Precision.HIGHEST


def _sc_gather_body(nw, bpw, nchunk,
                    idx_hbm, t0, t1, t2, t3, o0, o1, o2, o3,
                    idx_v, rows_v, sem):
    c_ax = lax.axis_index("c")
    s_ax = lax.axis_index("s")
    wid = s_ax * 2 + c_ax
    base = wid * bpw
    pltpu.sync_copy(idx_hbm.at[wid], idx_v)
    tabs = (t0, t1, t2, t3)
    outs = (o0, o1, o2, o3)
    prev = None
    for t in range(NT):
        for j in range(nchunk):
            d = pltpu.async_copy(
                tabs[t].at[idx_v.at[t, j]],
                rows_v.at[(t * nchunk + j) % 2],
                sem,
            )
            if prev is not None:
                pd, pt, pj = prev
                pd.wait()
                pltpu.sync_copy(
                    rows_v.at[(pt * nchunk + pj) % 2],
                    outs[pt].at[pl.ds(base + pj * CH, CH)])
            prev = (d, t, j)
    pd, pt, pj = prev
    pd.wait()
    pltpu.sync_copy(
        rows_v.at[(pt * nchunk + pj) % 2],
        outs[pt].at[pl.ds(base + pj * CH, CH)])


def _gather_rows(idx2, tp, t1, t2, t3):
    info = plsc.get_sparse_core_info()
    nw = info.num_cores * info.num_subcores
    bpw = B // nw
    nchunk = bpw // CH
    idx_r = jnp.transpose(idx2.reshape(NT, nw, nchunk, CH), (1, 0, 2, 3))
    mesh = plsc.VectorSubcoreMesh(core_axis_name="c", subcore_axis_name="s")
    body = functools.partial(_sc_gather_body, nw, bpw, nchunk)
    k = pl.kernel(
        body,
        out_type=[jax.ShapeDtypeStruct((B, 4 * H), jnp.float32)] * NT,
        mesh=mesh,
        scratch_types=[
            pltpu.VMEM((NT, nchunk, CH), jnp.int32),
            pltpu.VMEM((2, CH, 4 * H), jnp.float32),
            pltpu.SemaphoreType.DMA,
        ],
    )
    return k(idx_r, tp, t1, t2, t3)


BE = 8192


def _pack_body(src, out):
    parts = []
    for j in range(4):
        blk = src[:, j * (BE // 4):(j + 1) * (BE // 4)]
        parts.append(jnp.transpose(blk))
    out[:] = jnp.concatenate(parts, axis=1)


def _pack_table(t):
    tt = t.T
    V = t.shape[0]
    nblk = (V + BE - 1) // BE
    return pl.pallas_call(
        _pack_body,
        grid=(nblk,),
        in_specs=[pl.BlockSpec((H, BE), lambda i: (0, i))],
        out_specs=pl.BlockSpec((BE // 4, 4 * H), lambda i: (i, 0)),
        out_shape=jax.ShapeDtypeStruct((nblk * (BE // 4), 4 * H),
                                       jnp.float32),
        compiler_params=pltpu.CompilerParams(
            dimension_semantics=("parallel",)),
    )(tt)


def _tc_body(e0, e1, e2, e3, par, w1, b1, w2, b2, wpo, bpo, wppt, bpp,
             wv, bv, wvp, bvp, po_ref, v_ref):
    x1 = b1[:]
    for t, e in enumerate((e0, e1, e2, e3)):
        s = par[:, t:t + 1]
        lo = jnp.where(s == 1, e[:, H:2 * H], e[:, 0:H])
        hi = jnp.where(s == 3, e[:, 3 * H:4 * H], e[:, 2 * H:3 * H])
        sel = jnp.where(s >= 2, hi, lo)
        x1 = x1 + lax.dot_general(
            w1[:, t * H:(t + 1) * H], sel,
            (((1,), (1,)), ((), ())), precision=_HI)
    x1 = jnp.maximum(x1, 0.0)
    r2 = jnp.maximum(jnp.dot(w2[:], x1, precision=_HI) + b2[:], 0.0)
    rpo = jnp.maximum(jnp.dot(wpo[:], r2, precision=_HI) + bpo[:], 0.0)
    logits = lax.dot_general(wppt[:], rpo, (((0,), (0,)), ((), ())),
                             precision=_HI) + bpp[:]
    m = jnp.max(logits, axis=0, keepdims=True)
    lse = jnp.log(jnp.sum(jnp.exp(logits - m), axis=0, keepdims=True)) + m
    po_ref[:] = logits - lse
    rv = jnp.maximum(jnp.dot(wv[:], r2, precision=_HI) + bv[:], 0.0)
    v_ref[:] = jnp.tanh(jnp.sum(rv * wvp[:], axis=0, keepdims=True) + bvp[:])


def kernel(x_p, x_l, emb_p, emb_l1, emb_l2, emb_l3, W1, b1, W2, b2,
           Wpo, bpo, Wpp, bpp, Wv, bv, Wvp, bvp):
    P = Wpo.shape[0]
    V = Wpp.shape[0]
    tp, t1, t2 = (_pack_table(emb_p), _pack_table(emb_l1),
                  _pack_table(emb_l2))
    t3 = emb_l3.reshape(-1, 4 * H)
    idx = jnp.concatenate(
        [x_p.astype(jnp.int32), x_l.astype(jnp.int32)], axis=1)
    parp = (idx % BE) // (BE // 4)
    rowsp = (idx // BE) * (BE // 4) + (idx % (BE // 4))
    par = jnp.concatenate([parp[:, :3], (idx[:, 3:4] % 4)], axis=1)
    rows = jnp.concatenate([rowsp[:, :3], (idx[:, 3:4] // 4)], axis=1)
    e0, e1, e2, e3 = _gather_rows(rows.T, tp, t1, t2, t3)

    w1h = W1
    wppt = Wpp.T
    grid = (B // BM,)
    row = lambda i: (i, 0)
    col = lambda i: (0, i)
    full = lambda a: pl.BlockSpec(a.shape, lambda i: (0,) * a.ndim)
    b1c = b1.reshape(H, 1)
    b2c = b2.reshape(H, 1)
    bpoc = bpo.reshape(P, 1)
    bppc = bpp.reshape(V, 1)
    bvc = bv.reshape(P, 1)
    wvpc = Wvp.reshape(P, 1)
    bvpc = bvp.reshape(1, 1)
    e_spec = pl.BlockSpec((BM, 4 * H), row)
    pot, vt = pl.pallas_call(
        _tc_body,
        grid=grid,
        in_specs=[e_spec, e_spec, e_spec, e_spec,
                  pl.BlockSpec((BM, NT), row),
                  full(w1h), full(b1c), full(W2), full(b2c),
                  full(Wpo), full(bpoc), full(wppt), full(bppc),
                  full(Wv), full(bvc), full(wvpc), full(bvpc)],
        out_specs=[pl.BlockSpec((V, BM), col),
                   pl.BlockSpec((1, BM), col)],
        out_shape=[jax.ShapeDtypeStruct((V, B), jnp.float32),
                   jax.ShapeDtypeStruct((1, B), jnp.float32)],
        compiler_params=pltpu.CompilerParams(
            dimension_semantics=("arbitrary",)),
    )(e0, e1, e2, e3, par, w1h, b1c, W2, b2c, Wpo, bpoc, wppt, bppc,
      Wv, bvc, wvpc, bvpc)
    return (pot.T, vt.T)


# speedup vs baseline: 1.3160x; 1.3160x over previous
import functools

import jax
import jax.numpy as jnp
from jax import lax
from jax.experimental import pallas as pl
from jax.experimental.pallas import tpu as pltpu
from jax.experimental.pallas import tpu_sc as plsc

B = 16384
H = 64
NT = 4
CH = 128
BM = 512

_HI = jax.lax.Precision.HIGHEST


def _sc_gather_body(nw, bpw, nchunk,
                    idx_hbm, t0, t1, t2, t3, o0, o1, o2, o3,
                    idx_v, rows_v, sem):
    c_ax = lax.axis_index("c")
    s_ax = lax.axis_index("s")
    wid = s_ax * 2 + c_ax
    base = wid * bpw
    pltpu.sync_copy(idx_hbm.at[wid], idx_v)
    tabs = (t0, t1, t2, t3)
    outs = (o0, o1, o2, o3)
    prev = None
    for t in range(NT):
        for j in range(nchunk):
            d = pltpu.async_copy(
                tabs[t].at[idx_v.at[t, j]],
                rows_v.at[(t * nchunk + j) % 2],
                sem,
            )
            if prev is not None:
                pd, pt, pj = prev
                pd.wait()
                pltpu.sync_copy(
                    rows_v.at[(pt * nchunk + pj) % 2],
                    outs[pt].at[pl.ds(base + pj * CH, CH)])
            prev = (d, t, j)
    pd, pt, pj = prev
    pd.wait()
    pltpu.sync_copy(
        rows_v.at[(pt * nchunk + pj) % 2],
        outs[pt].at[pl.ds(base + pj * CH, CH)])


def _gather_rows(idx2, tp, t1, t2, t3):
    info = plsc.get_sparse_core_info()
    nw = info.num_cores * info.num_subcores
    bpw = B // nw
    nchunk = bpw // CH
    idx_r = jnp.transpose(idx2.reshape(NT, nw, nchunk, CH), (1, 0, 2, 3))
    mesh = plsc.VectorSubcoreMesh(core_axis_name="c", subcore_axis_name="s")
    body = functools.partial(_sc_gather_body, nw, bpw, nchunk)
    k = pl.kernel(
        body,
        out_type=[jax.ShapeDtypeStruct((B, 4 * H), jnp.float32)] * NT,
        mesh=mesh,
        scratch_types=[
            pltpu.VMEM((NT, nchunk, CH), jnp.int32),
            pltpu.VMEM((2, CH, 4 * H), jnp.float32),
            pltpu.SemaphoreType.DMA,
        ],
    )
    return k(idx_r, tp, t1, t2, t3)


BE = 16384


def _pack_body(src, out):
    parts = []
    for j in range(4):
        blk = src[:, j * (BE // 4):(j + 1) * (BE // 4)]
        parts.append(jnp.transpose(blk))
    out[:] = jnp.concatenate(parts, axis=1)


def _pack_table(t):
    tt = t.T
    V = t.shape[0]
    nblk = (V + BE - 1) // BE
    return pl.pallas_call(
        _pack_body,
        grid=(nblk,),
        in_specs=[pl.BlockSpec((H, BE), lambda i: (0, i))],
        out_specs=pl.BlockSpec((BE // 4, 4 * H), lambda i: (i, 0)),
        out_shape=jax.ShapeDtypeStruct((nblk * (BE // 4), 4 * H),
                                       jnp.float32),
        compiler_params=pltpu.CompilerParams(
            dimension_semantics=("parallel",)),
    )(tt)


def _tc_body(e0, e1, e2, e3, par, w1, b1, w2, b2, wpo, bpo, wppt, bpp,
             wv, bv, wvp, bvp, po_ref, v_ref):
    x1 = b1[:]
    for t, e in enumerate((e0, e1, e2, e3)):
        s = par[:, t:t + 1]
        lo = jnp.where(s == 1, e[:, H:2 * H], e[:, 0:H])
        hi = jnp.where(s == 3, e[:, 3 * H:4 * H], e[:, 2 * H:3 * H])
        sel = jnp.where(s >= 2, hi, lo)
        x1 = x1 + lax.dot_general(
            w1[:, t * H:(t + 1) * H], sel,
            (((1,), (1,)), ((), ())), precision=_HI)
    x1 = jnp.maximum(x1, 0.0)
    r2 = jnp.maximum(jnp.dot(w2[:], x1, precision=_HI) + b2[:], 0.0)
    rpo = jnp.maximum(jnp.dot(wpo[:], r2, precision=_HI) + bpo[:], 0.0)
    logits = lax.dot_general(wppt[:], rpo, (((0,), (0,)), ((), ())),
                             precision=_HI) + bpp[:]
    m = jnp.max(logits, axis=0, keepdims=True)
    lse = jnp.log(jnp.sum(jnp.exp(logits - m), axis=0, keepdims=True)) + m
    po_ref[:] = logits - lse
    rv = jnp.maximum(jnp.dot(wv[:], r2, precision=_HI) + bv[:], 0.0)
    v_ref[:] = jnp.tanh(jnp.sum(rv * wvp[:], axis=0, keepdims=True) + bvp[:])


def kernel(x_p, x_l, emb_p, emb_l1, emb_l2, emb_l3, W1, b1, W2, b2,
           Wpo, bpo, Wpp, bpp, Wv, bv, Wvp, bvp):
    P = Wpo.shape[0]
    V = Wpp.shape[0]
    tp, t1, t2, t3 = (_pack_table(emb_p), _pack_table(emb_l1),
                      _pack_table(emb_l2), _pack_table(emb_l3))
    idx = jnp.concatenate(
        [x_p.astype(jnp.int32), x_l.astype(jnp.int32)], axis=1)
    par = (idx % BE) // (BE // 4)
    rows = (idx // BE) * (BE // 4) + (idx % (BE // 4))
    e0, e1, e2, e3 = _gather_rows(rows.T, tp, t1, t2, t3)

    w1h = W1
    wppt = Wpp.T
    grid = (B // BM,)
    row = lambda i: (i, 0)
    col = lambda i: (0, i)
    full = lambda a: pl.BlockSpec(a.shape, lambda i: (0,) * a.ndim)
    b1c = b1.reshape(H, 1)
    b2c = b2.reshape(H, 1)
    bpoc = bpo.reshape(P, 1)
    bppc = bpp.reshape(V, 1)
    bvc = bv.reshape(P, 1)
    wvpc = Wvp.reshape(P, 1)
    bvpc = bvp.reshape(1, 1)
    e_spec = pl.BlockSpec((BM, 4 * H), row)
    pot, vt = pl.pallas_call(
        _tc_body,
        grid=grid,
        in_specs=[e_spec, e_spec, e_spec, e_spec,
                  pl.BlockSpec((BM, NT), row),
                  full(w1h), full(b1c), full(W2), full(b2c),
                  full(Wpo), full(bpoc), full(wppt), full(bppc),
                  full(Wv), full(bvc), full(wvpc), full(bvpc)],
        out_specs=[pl.BlockSpec((V, BM), col),
                   pl.BlockSpec((1, BM), col)],
        out_shape=[jax.ShapeDtypeStruct((V, B), jnp.float32),
                   jax.ShapeDtypeStruct((1, B), jnp.float32)],
        compiler_params=pltpu.CompilerParams(
            dimension_semantics=("arbitrary",)),
    )(e0, e1, e2, e3, par, w1h, b1c, W2, b2c, Wpo, bpoc, wppt, bppc,
      Wv, bvc, wvpc, bvpc)
    return (pot.T, vt.T)


# speedup vs baseline: 1.3759x; 1.0456x over previous
import functools

import jax
import jax.numpy as jnp
from jax import lax
from jax.experimental import pallas as pl
from jax.experimental.pallas import tpu as pltpu
from jax.experimental.pallas import tpu_sc as plsc

B = 16384
H = 64
NT = 4
CH = 128
BM = 512

_HI = jax.lax.Precision.HIGHEST


def _sc_gather_body(nw, bpw, nchunk,
                    idx_hbm, t0, t1, t2, t3, o0, o1, o2, o3,
                    idx_v, rows_v, sem):
    c_ax = lax.axis_index("c")
    s_ax = lax.axis_index("s")
    wid = s_ax * 2 + c_ax
    base = wid * bpw
    pltpu.sync_copy(idx_hbm.at[wid], idx_v)
    tabs = (t0, t1, t2, t3)
    outs = (o0, o1, o2, o3)
    prev = None
    for t in range(NT):
        for j in range(nchunk):
            d = pltpu.async_copy(
                tabs[t].at[idx_v.at[t, j]],
                rows_v.at[(t * nchunk + j) % 2],
                sem,
            )
            if prev is not None:
                pd, pt, pj = prev
                pd.wait()
                pltpu.sync_copy(
                    rows_v.at[(pt * nchunk + pj) % 2],
                    outs[pt].at[pl.ds(base + pj * CH, CH)])
            prev = (d, t, j)
    pd, pt, pj = prev
    pd.wait()
    pltpu.sync_copy(
        rows_v.at[(pt * nchunk + pj) % 2],
        outs[pt].at[pl.ds(base + pj * CH, CH)])


def _gather_rows(idx2, tp, t1, t2, t3):
    info = plsc.get_sparse_core_info()
    nw = info.num_cores * info.num_subcores
    bpw = B // nw
    nchunk = bpw // CH
    idx_r = jnp.transpose(idx2.reshape(NT, nw, nchunk, CH), (1, 0, 2, 3))
    mesh = plsc.VectorSubcoreMesh(core_axis_name="c", subcore_axis_name="s")
    body = functools.partial(_sc_gather_body, nw, bpw, nchunk)
    k = pl.kernel(
        body,
        out_type=[jax.ShapeDtypeStruct((B, 4 * H), jnp.float32)] * NT,
        mesh=mesh,
        scratch_types=[
            pltpu.VMEM((NT, nchunk, CH), jnp.int32),
            pltpu.VMEM((2, CH, 4 * H), jnp.float32),
            pltpu.SemaphoreType.DMA,
        ],
    )
    return k(idx_r, tp, t1, t2, t3)


BE = 32768


def _pack_body(src, out):
    parts = []
    for j in range(4):
        blk = src[:, j * (BE // 4):(j + 1) * (BE // 4)]
        parts.append(jnp.transpose(blk))
    out[:] = jnp.concatenate(parts, axis=1)


def _pack_table(t):
    tt = t.T
    V = t.shape[0]
    nblk = (V + BE - 1) // BE
    return pl.pallas_call(
        _pack_body,
        grid=(nblk,),
        in_specs=[pl.BlockSpec((H, BE), lambda i: (0, i))],
        out_specs=pl.BlockSpec((BE // 4, 4 * H), lambda i: (i, 0)),
        out_shape=jax.ShapeDtypeStruct((nblk * (BE // 4), 4 * H),
                                       jnp.float32),
        compiler_params=pltpu.CompilerParams(
            dimension_semantics=("parallel",)),
    )(tt)


def _tc_body(e0, e1, e2, e3, par, w1, b1, w2, b2, wpo, bpo, wppt, bpp,
             wv, bv, wvp, bvp, po_ref, v_ref):
    x1 = b1[:]
    for t, e in enumerate((e0, e1, e2, e3)):
        s = par[:, t:t + 1]
        lo = jnp.where(s == 1, e[:, H:2 * H], e[:, 0:H])
        hi = jnp.where(s == 3, e[:, 3 * H:4 * H], e[:, 2 * H:3 * H])
        sel = jnp.where(s >= 2, hi, lo)
        x1 = x1 + lax.dot_general(
            w1[:, t * H:(t + 1) * H], sel,
            (((1,), (1,)), ((), ())), precision=_HI)
    x1 = jnp.maximum(x1, 0.0)
    r2 = jnp.maximum(jnp.dot(w2[:], x1, precision=_HI) + b2[:], 0.0)
    rpo = jnp.maximum(jnp.dot(wpo[:], r2, precision=_HI) + bpo[:], 0.0)
    logits = lax.dot_general(wppt[:], rpo, (((0,), (0,)), ((), ())),
                             precision=_HI) + bpp[:]
    m = jnp.max(logits, axis=0, keepdims=True)
    lse = jnp.log(jnp.sum(jnp.exp(logits - m), axis=0, keepdims=True)) + m
    po_ref[:] = logits - lse
    rv = jnp.maximum(jnp.dot(wv[:], r2, precision=_HI) + bv[:], 0.0)
    v_ref[:] = jnp.tanh(jnp.sum(rv * wvp[:], axis=0, keepdims=True) + bvp[:])


def kernel(x_p, x_l, emb_p, emb_l1, emb_l2, emb_l3, W1, b1, W2, b2,
           Wpo, bpo, Wpp, bpp, Wv, bv, Wvp, bvp):
    P = Wpo.shape[0]
    V = Wpp.shape[0]
    tp, t1, t2, t3 = (_pack_table(emb_p), _pack_table(emb_l1),
                      _pack_table(emb_l2), _pack_table(emb_l3))
    idx = jnp.concatenate(
        [x_p.astype(jnp.int32), x_l.astype(jnp.int32)], axis=1)
    par = (idx % BE) // (BE // 4)
    rows = (idx // BE) * (BE // 4) + (idx % (BE // 4))
    e0, e1, e2, e3 = _gather_rows(rows.T, tp, t1, t2, t3)

    w1h = W1
    wppt = Wpp.T
    grid = (B // BM,)
    row = lambda i: (i, 0)
    col = lambda i: (0, i)
    full = lambda a: pl.BlockSpec(a.shape, lambda i: (0,) * a.ndim)
    b1c = b1.reshape(H, 1)
    b2c = b2.reshape(H, 1)
    bpoc = bpo.reshape(P, 1)
    bppc = bpp.reshape(V, 1)
    bvc = bv.reshape(P, 1)
    wvpc = Wvp.reshape(P, 1)
    bvpc = bvp.reshape(1, 1)
    e_spec = pl.BlockSpec((BM, 4 * H), row)
    pot, vt = pl.pallas_call(
        _tc_body,
        grid=grid,
        in_specs=[e_spec, e_spec, e_spec, e_spec,
                  pl.BlockSpec((BM, NT), row),
                  full(w1h), full(b1c), full(W2), full(b2c),
                  full(Wpo), full(bpoc), full(wppt), full(bppc),
                  full(Wv), full(bvc), full(wvpc), full(bvpc)],
        out_specs=[pl.BlockSpec((V, BM), col),
                   pl.BlockSpec((1, BM), col)],
        out_shape=[jax.ShapeDtypeStruct((V, B), jnp.float32),
                   jax.ShapeDtypeStruct((1, B), jnp.float32)],
        compiler_params=pltpu.CompilerParams(
            dimension_semantics=("arbitrary",)),
    )(e0, e1, e2, e3, par, w1h, b1c, W2, b2c, Wpo, bpoc, wppt, bppc,
      Wv, bvc, wvpc, bvpc)
    return (pot.T, vt.T)
